# baseline (speedup 1.0000x reference)
import functools

import jax
import jax.numpy as jnp
from jax import lax
from jax.experimental import pallas as pl
from jax.experimental.pallas import tpu as pltpu
from jax.experimental.pallas import tpu_sc as plsc

N = 50000
E = 800000
HID = 64
HALF = 32

NTILES = 16
LANES = 128
KG = 2
E_PAD = 802816
EROWS = E_PAD // LANES
RPT = EROWS // NTILES
NCHUNK = RPT // KG
R_PAD = 50000
RNODE = R_PAD // NTILES
DROWS = 125

BN = 2000


def _mlp_in(beta, degree, W_in, b_in):
    def body(b_ref, d_ref, w_ref, bias_ref, h0_ref, h1_ref):
        b = b_ref[...]
        d = d_ref[...]
        w = w_ref[...]
        bias = bias_ref[...]
        h = b * w[0:1, :] + (b * b) * w[1:2, :] + d * w[2:3, :] + bias
        h = jnp.maximum(h, 0.0)
        h0_ref[...] = h[:, :HALF]
        h1_ref[...] = h[:, HALF:]

    return pl.pallas_call(
        body,
        grid=(N // BN,),
        in_specs=[
            pl.BlockSpec((BN, 1), lambda i: (i, 0)),
            pl.BlockSpec((BN, 1), lambda i: (i, 0)),
            pl.BlockSpec((3, HID), lambda i: (0, 0)),
            pl.BlockSpec((1, HID), lambda i: (0, 0)),
        ],
        out_specs=[
            pl.BlockSpec((BN, HALF), lambda i: (i, 0)),
            pl.BlockSpec((BN, HALF), lambda i: (i, 0)),
        ],
        out_shape=[
            jax.ShapeDtypeStruct((N, HALF), jnp.float32),
            jax.ShapeDtypeStruct((N, HALF), jnp.float32),
        ],
    )(beta, degree, W_in, b_in.reshape(1, HID))


def _mlp_out(big, wbig, W_out, b_out):
    def body(x_ref, wb_ref, wo_ref, bo_ref, g_ref):
        h2 = jnp.maximum(
            jnp.dot(x_ref[...], wb_ref[...],
                    preferred_element_type=jnp.float32), 0.0)
        z = jnp.dot(h2, wo_ref[...],
                    preferred_element_type=jnp.float32) + bo_ref[...]
        g_ref[...] = jnp.maximum(z, 0.0) + jnp.log(1.0 + jnp.exp(-jnp.abs(z)))

    full = lambda shape: pl.BlockSpec(shape, lambda i: (0, 0))
    return pl.pallas_call(
        body,
        grid=(N // BN,),
        in_specs=[
            pl.BlockSpec((BN, 2 * HID), lambda i: (i, 0)),
            full((2 * HID, HID)),
            full((HID, 1)),
            full((1, 1)),
        ],
        out_specs=pl.BlockSpec((BN, 1), lambda i: (i, 0)),
        out_shape=jax.ShapeDtypeStruct((N, 1), jnp.float32),
    )(big, wbig, W_out, b_out.reshape(1, 1))


def _spmm2(h0, h1, edata):
    mesh = plsc.VectorSubcoreMesh(core_axis_name="c", subcore_axis_name="s")
    out = jax.ShapeDtypeStruct((R_PAD, HALF), jnp.float32)
    big = jax.ShapeDtypeStruct((R_PAD, 4 * HALF), jnp.float32)

    @functools.partial(
        pl.kernel,
        mesh=mesh,
        out_type=[big, out, out],
        compiler_params=pltpu.CompilerParams(
            use_tc_tiling_on_sc=False, needs_layout_passes=False),
        scratch_types=[
            pltpu.VMEM((3, KG, LANES), jnp.int32),
            pltpu.VMEM((3, KG, LANES), jnp.int32),
            pltpu.VMEM((KG, LANES, HALF), jnp.float32),
            pltpu.VMEM((KG, LANES, HALF), jnp.float32),
            pltpu.VMEM((DROWS, HALF), jnp.float32),
            pltpu.VMEM((DROWS, HALF), jnp.float32),
            pltpu.VMEM_SHARED((R_PAD, HALF), jnp.float32),
            pltpu.SemaphoreType.DMA,
            pltpu.SemaphoreType.DMA,
            pltpu.SemaphoreType.DMA,
            pltpu.SemaphoreType.DMA,
        ],
    )
    def sc(h0r, h1r, edr, bigr, ah0r, ah1r,
           eb0, eb1, rw0, rw1, stage, zbuf, acc, sem0, sem1, ssem0, ssem1):
        c = lax.axis_index("c")
        s = lax.axis_index("s")
        zvec = jnp.zeros((16,), jnp.float32)

        def zb_init(r, carry):
            zbuf[r, pl.ds(0, 16)] = zvec
            zbuf[r, pl.ds(16, 16)] = zvec
            return carry

        lax.fori_loop(0, DROWS, zb_init, 0)

        for q in range(RNODE // DROWS):
            pltpu.sync_copy(zbuf, acc.at[pl.ds(s * RNODE + q * DROWS, DROWS)])
        plsc.subcore_barrier()

        bufs = ((eb0, rw0, sem0, ssem0), (eb1, rw1, sem1, ssem1))

        def edge_pass(table):
            def load_chunk(ci, eb):
                base = s * RPT + ci * KG
                for i in range(3):
                    pltpu.sync_copy(edr.at[i, pl.ds(base, KG)], eb.at[i])

            def fire(eb, rw, sem):
                for j in range(KG):
                    pltpu.async_copy(table.at[eb.at[0, j]], rw.at[j], sem)

            def wait_g(eb, rw, sem):
                for j in range(KG):
                    pltpu.make_async_copy(
                        table.at[eb.at[0, j]], rw.at[j], sem).wait()

            def scale(eb, rw):
                for j in range(KG):
                    def body(b, carry2):
                        w16 = plsc.bitcast(eb[2, j, pl.ds(b * 16, 16)],
                                           jnp.float32)
                        for u in range(16):
                            e = b * 16 + u
                            wv = w16[u]
                            rw[j, e, pl.ds(0, 16)] = rw[j, e, pl.ds(0, 16)] * wv
                            rw[j, e, pl.ds(16, 16)] = rw[j, e, pl.ds(16, 16)] * wv
                        return carry2
                    lax.fori_loop(0, LANES // 16, body, 0)

            def fire_s(eb, rw, ssem):
                for j in range(KG):
                    pltpu.async_copy(rw.at[j], acc.at[eb.at[1, j]], ssem,
                                     add=True)

            def wait_s(eb, rw, ssem):
                for j in range(KG):
                    pltpu.make_async_copy(rw.at[j], acc.at[eb.at[1, j]],
                                          ssem).wait()

            load_chunk(0, eb0)
            fire(eb0, rw0, sem0)

            def pair(p, carry):
                for half in range(2):
                    ci = p * 2 + half
                    eb, rw, sem, ssem = bufs[half]
                    ebn, rwn, semn, ssemn = bufs[1 - half]

                    @pl.when(ci + 1 < NCHUNK)
                    def _():
                        @pl.when(ci >= 1)
                        def _():
                            wait_s(ebn, rwn, ssemn)
                        load_chunk(ci + 1, ebn)
                        fire(ebn, rwn, semn)

                    wait_g(eb, rw, sem)
                    scale(eb, rw)
                    fire_s(eb, rw, ssem)
                return carry

            lax.fori_loop(0, NCHUNK // 2, pair, 0)
            wait_s(eb0, rw0, ssem0)
            wait_s(eb1, rw1, ssem1)

        def dump(cbase, table_ref, rezero):
            for q in range(RNODE // DROWS):
                r0 = s * RNODE + q * DROWS
                pltpu.sync_copy(acc.at[pl.ds(r0, DROWS)], stage)
                pltpu.sync_copy(stage,
                                bigr.at[pl.ds(r0, DROWS), pl.ds(cbase, HALF)])
                if table_ref is not None:
                    pltpu.sync_copy(stage, table_ref.at[pl.ds(r0, DROWS)])
                if rezero:
                    pltpu.sync_copy(zbuf, acc.at[pl.ds(r0, DROWS)])

        @pl.when(c == 0)
        def _():
            edge_pass(h0r)

        @pl.when(c == 1)
        def _():
            edge_pass(h1r)

        plsc.subcore_barrier()

        @pl.when(c == 0)
        def _():
            dump(0 * HALF, ah0r, True)

        @pl.when(c == 1)
        def _():
            dump(1 * HALF, ah1r, True)

        plsc.subcore_barrier()

        @pl.when(c == 0)
        def _():
            edge_pass(ah0r)

        @pl.when(c == 1)
        def _():
            edge_pass(ah1r)

        plsc.subcore_barrier()

        @pl.when(c == 0)
        def _():
            dump(2 * HALF, None, False)

        @pl.when(c == 1)
        def _():
            dump(3 * HALF, None, False)

    return sc(h0, h1, edata)[0]


def kernel(beta, degree, edge_index, edge_weight, W_in, b_in, W_mp1, W_mp2,
           W_out, b_out):
    pad = E_PAD - E
    srcm = jnp.pad(edge_index[0], (0, pad)).reshape(EROWS, LANES)
    dstm = jnp.pad(edge_index[1], (0, pad)).reshape(EROWS, LANES)
    wm = lax.bitcast_convert_type(
        jnp.pad(edge_weight, (0, pad)).reshape(EROWS, LANES), jnp.int32)
    edata = jnp.stack([srcm, dstm, wm])

    h0, h1 = _mlp_in(beta, degree, W_in, b_in)
    big = _spmm2(h0, h1, edata)
    wbig = jnp.concatenate([W_mp1, W_mp2], axis=0)
    return _mlp_out(big, wbig, W_out, b_out)


# speedup vs baseline: 1.4016x; 1.4016x over previous
import functools

import jax
import jax.numpy as jnp
from jax import lax
from jax.experimental import pallas as pl
from jax.experimental.pallas import tpu as pltpu
from jax.experimental.pallas import tpu_sc as plsc

N = 50000
E = 800000
HID = 64
HALF = 32

NTILES = 16
LANES = 128
KG = 2
E_PAD = 802816
EROWS = E_PAD // LANES
RPT = EROWS // NTILES
NCHUNK = RPT // KG
R_PAD = 50000
RNODE = R_PAD // NTILES
DROWS = 125

BN = 2000


def _mlp_in(beta, degree, W_in, b_in):
    def body(b_ref, d_ref, w_ref, bias_ref, h0_ref, h1_ref):
        b = b_ref[...]
        d = d_ref[...]
        w = w_ref[...]
        bias = bias_ref[...]
        h = b * w[0:1, :] + (b * b) * w[1:2, :] + d * w[2:3, :] + bias
        h = jnp.maximum(h, 0.0)
        h0_ref[...] = h[:, :HALF]
        h1_ref[...] = h[:, HALF:]

    return pl.pallas_call(
        body,
        grid=(N // BN,),
        in_specs=[
            pl.BlockSpec((BN, 1), lambda i: (i, 0)),
            pl.BlockSpec((BN, 1), lambda i: (i, 0)),
            pl.BlockSpec((3, HID), lambda i: (0, 0)),
            pl.BlockSpec((1, HID), lambda i: (0, 0)),
        ],
        out_specs=[
            pl.BlockSpec((BN, HALF), lambda i: (i, 0)),
            pl.BlockSpec((BN, HALF), lambda i: (i, 0)),
        ],
        out_shape=[
            jax.ShapeDtypeStruct((N, HALF), jnp.float32),
            jax.ShapeDtypeStruct((N, HALF), jnp.float32),
        ],
    )(beta, degree, W_in, b_in.reshape(1, HID))


def _mlp_out(a0, a1, b0, b1, w1a, w1b, w2a, w2b, wo, b_out):
    NQ = N // 4

    def body(a0_ref, a1_ref, b0_ref, b1_ref, w1a_ref, w1b_ref, w2a_ref,
             w2b_ref, wo_ref, bo_ref, g_ref):
        y = (
            jnp.dot(a0_ref[...], w1a_ref[...], preferred_element_type=jnp.float32)
            + jnp.dot(a1_ref[...], w1b_ref[...], preferred_element_type=jnp.float32)
            + jnp.dot(b0_ref[...], w2a_ref[...], preferred_element_type=jnp.float32)
            + jnp.dot(b1_ref[...], w2b_ref[...], preferred_element_type=jnp.float32)
        )
        h2 = jnp.maximum(y, 0.0)
        z = jnp.dot(h2, wo_ref[...], preferred_element_type=jnp.float32) + bo_ref[...]
        g_ref[...] = jnp.maximum(z, 0.0) + jnp.log(1.0 + jnp.exp(-jnp.abs(z)))

    full = lambda shape: pl.BlockSpec(shape, lambda: (0, 0))
    return pl.pallas_call(
        body,
        in_specs=[
            full((NQ, 128)),
            full((NQ, 128)),
            full((NQ, 128)),
            full((NQ, 128)),
            full((128, 256)),
            full((128, 256)),
            full((128, 256)),
            full((128, 256)),
            full((256, 4)),
            full((1, 1)),
        ],
        out_specs=full((NQ, 4)),
        out_shape=jax.ShapeDtypeStruct((NQ, 4), jnp.float32),
    )(a0, a1, b0, b1, w1a, w1b, w2a, w2b, wo, b_out.reshape(1, 1))


def _spmm2(h0, h1, edata):
    mesh = plsc.VectorSubcoreMesh(core_axis_name="c", subcore_axis_name="s")
    out = jax.ShapeDtypeStruct((R_PAD, HALF), jnp.float32)

    @functools.partial(
        pl.kernel,
        mesh=mesh,
        out_type=[out, out, out, out],
        compiler_params=pltpu.CompilerParams(
            use_tc_tiling_on_sc=False, needs_layout_passes=False),
        scratch_types=[
            pltpu.VMEM((KG, 3, LANES), jnp.int32),
            pltpu.VMEM((KG, 3, LANES), jnp.int32),
            pltpu.VMEM((KG, LANES, HALF), jnp.float32),
            pltpu.VMEM((KG, LANES, HALF), jnp.float32),
            pltpu.VMEM((DROWS, HALF), jnp.float32),
            pltpu.VMEM((DROWS, HALF), jnp.float32),
            pltpu.VMEM_SHARED((R_PAD, HALF), jnp.float32),
            pltpu.SemaphoreType.DMA,
            pltpu.SemaphoreType.DMA,
            pltpu.SemaphoreType.DMA,
            pltpu.SemaphoreType.DMA,
        ],
    )
    def sc(h0r, h1r, edr, ah0r, ah1r, a2h0r, a2h1r,
           eb0, eb1, rw0, rw1, stage, zbuf, acc, sem0, sem1, ssem0, ssem1):
        c = lax.axis_index("c")
        s = lax.axis_index("s")
        zvec = jnp.zeros((16,), jnp.float32)

        def zb_init(r, carry):
            zbuf[r, pl.ds(0, 16)] = zvec
            zbuf[r, pl.ds(16, 16)] = zvec
            return carry

        lax.fori_loop(0, DROWS, zb_init, 0)

        for q in range(RNODE // DROWS):
            pltpu.sync_copy(zbuf, acc.at[pl.ds(s * RNODE + q * DROWS, DROWS)])
        plsc.subcore_barrier()

        bufs = ((eb0, rw0, sem0, ssem0), (eb1, rw1, sem1, ssem1))

        def edge_pass(table):
            def load_chunk(ci, eb):
                base = s * RPT + ci * KG
                pltpu.sync_copy(edr.at[pl.ds(base, KG)], eb)

            def fire(eb, rw, sem):
                for j in range(KG):
                    pltpu.async_copy(table.at[eb.at[j, 0]], rw.at[j], sem)

            def wait_g(eb, rw, sem):
                for j in range(KG):
                    pltpu.make_async_copy(
                        table.at[eb.at[j, 0]], rw.at[j], sem).wait()

            def scale(eb, rw):
                for j in range(KG):
                    def body(b, carry2):
                        w16 = plsc.bitcast(eb[j, 2, pl.ds(b * 16, 16)],
                                           jnp.float32)
                        for u in range(16):
                            e = b * 16 + u
                            wv = w16[u]
                            rw[j, e, pl.ds(0, 16)] = rw[j, e, pl.ds(0, 16)] * wv
                            rw[j, e, pl.ds(16, 16)] = rw[j, e, pl.ds(16, 16)] * wv
                        return carry2
                    lax.fori_loop(0, LANES // 16, body, 0)

            def fire_s(eb, rw, ssem):
                for j in range(KG):
                    pltpu.async_copy(rw.at[j], acc.at[eb.at[j, 1]], ssem,
                                     add=True)

            def wait_s(eb, rw, ssem):
                for j in range(KG):
                    pltpu.make_async_copy(rw.at[j], acc.at[eb.at[j, 1]],
                                          ssem).wait()

            load_chunk(0, eb0)
            fire(eb0, rw0, sem0)

            def pair(p, carry):
                for half in range(2):
                    ci = p * 2 + half
                    eb, rw, sem, ssem = bufs[half]
                    ebn, rwn, semn, ssemn = bufs[1 - half]

                    @pl.when(ci + 1 < NCHUNK)
                    def _():
                        @pl.when(ci >= 1)
                        def _():
                            wait_s(ebn, rwn, ssemn)
                        load_chunk(ci + 1, ebn)
                        fire(ebn, rwn, semn)

                    wait_g(eb, rw, sem)
                    scale(eb, rw)
                    fire_s(eb, rw, ssem)
                return carry

            lax.fori_loop(0, NCHUNK // 2, pair, 0)
            wait_s(eb0, rw0, ssem0)
            wait_s(eb1, rw1, ssem1)

        def dump(out_ref):
            for q in range(RNODE // DROWS):
                r0 = s * RNODE + q * DROWS
                pltpu.sync_copy(acc.at[pl.ds(r0, DROWS)], stage)
                pltpu.sync_copy(stage, out_ref.at[pl.ds(r0, DROWS)])
                pltpu.sync_copy(zbuf, acc.at[pl.ds(r0, DROWS)])

        @pl.when(c == 0)
        def _():
            edge_pass(h0r)

        @pl.when(c == 1)
        def _():
            edge_pass(h1r)

        plsc.subcore_barrier()

        @pl.when(c == 0)
        def _():
            dump(ah0r)

        @pl.when(c == 1)
        def _():
            dump(ah1r)

        plsc.subcore_barrier()

        @pl.when(c == 0)
        def _():
            edge_pass(ah0r)

        @pl.when(c == 1)
        def _():
            edge_pass(ah1r)

        plsc.subcore_barrier()

        @pl.when(c == 0)
        def _():
            dump(a2h0r)

        @pl.when(c == 1)
        def _():
            dump(a2h1r)

    return sc(h0, h1, edata)


def kernel(beta, degree, edge_index, edge_weight, W_in, b_in, W_mp1, W_mp2,
           W_out, b_out):
    pad = E_PAD - E
    srcm = jnp.pad(edge_index[0], (0, pad)).reshape(EROWS, LANES)
    dstm = jnp.pad(edge_index[1], (0, pad)).reshape(EROWS, LANES)
    wm = lax.bitcast_convert_type(
        jnp.pad(edge_weight, (0, pad)).reshape(EROWS, LANES), jnp.int32)
    edata = jnp.stack([srcm, dstm, wm], axis=1)

    h0, h1 = _mlp_in(beta, degree, W_in, b_in)
    ah0, ah1, a2h0, a2h1 = _spmm2(h0, h1, edata)
    eye4 = jnp.eye(4, dtype=jnp.float32)
    bd = lambda w: jnp.kron(eye4, w)
    g4 = _mlp_out(
        ah0.reshape(N // 4, 128), ah1.reshape(N // 4, 128),
        a2h0.reshape(N // 4, 128), a2h1.reshape(N // 4, 128),
        bd(W_mp1[:HALF]), bd(W_mp1[HALF:]),
        bd(W_mp2[:HALF]), bd(W_mp2[HALF:]),
        bd(W_out), b_out,
    )
    return g4.reshape(N, 1)
